# Initial kernel scaffold; baseline (speedup 1.0000x reference)
import jax
import jax.numpy as jnp
from jax.experimental import pallas as pl


def kernel(x, edge_index, node_id, W1, b1, W2, b2, Wi1, bi1, Wi2, bi2):
    raise NotImplementedError("write your pallas kernel here")


# speedup vs baseline: 6.6176x; 6.6176x over previous
import functools

import jax
import jax.numpy as jnp
from jax import lax
from jax.experimental import pallas as pl
from jax.experimental.pallas import tpu as pltpu
from jax.experimental.pallas import tpu_sc as plsc

N = 10000
E = 320000
D = 128
NID = 1000

R = 10240
DUMMY = 10232
NID_PAD = 10112
CHUNK = 128
NCHUNKS = E // CHUNK
NW = 32
ROWS_PER_TILE = R // 16
NID_TOT = 1024


def _sc_agg_body(x_hbm, src_hbm, dst_hbm,
                 agg_out,
                 src_v, dst_v, rows_v, zb,
                 agg_sh):
    cid = lax.axis_index("c")
    sid = lax.axis_index("s")
    w = sid * 2 + cid

    for i in range(16):
        for k in range(D // 16):
            zb[i, pl.ds(k * 16, 16)] = jnp.zeros((16,), jnp.float32)
    row0 = sid * ROWS_PER_TILE
    for t in range(ROWS_PER_TILE // 16):
        pltpu.sync_copy(zb, agg_sh.at[pl.ds(row0 + t * 16, 16)])
    plsc.subcore_barrier()

    lo = (w * NCHUNKS) // NW
    hi = ((w + 1) * NCHUNKS) // NW

    def chunk_body(ci, _):
        base = ci * CHUNK
        pltpu.sync_copy(src_hbm.at[pl.ds(base, CHUNK)], src_v)
        pltpu.sync_copy(dst_hbm.at[pl.ds(base, CHUNK)], dst_v)
        for k in range(CHUNK // 16):
            sl = pl.ds(k * 16, 16)
            s = src_v[sl]
            d = dst_v[sl]
            dst_v[sl] = jnp.where(s == d, jnp.int32(DUMMY), d)
        pltpu.sync_copy(x_hbm.at[src_v], rows_v)
        pltpu.sync_copy(rows_v, agg_sh.at[dst_v], add=True)
        return 0

    lax.fori_loop(lo, hi, chunk_body, 0)
    plsc.subcore_barrier()

    pltpu.sync_copy(agg_sh.at[pl.ds(row0, ROWS_PER_TILE)],
                    agg_out.at[cid, pl.ds(row0, ROWS_PER_TILE)])


def _sc_aggregate(x, src, dst):
    mesh = plsc.VectorSubcoreMesh(core_axis_name="c", subcore_axis_name="s")
    f = functools.partial(
        pl.kernel,
        mesh=mesh,
        out_type=jax.ShapeDtypeStruct((2, R, D), jnp.float32),
        scratch_types=[
            pltpu.VMEM((CHUNK,), jnp.int32),
            pltpu.VMEM((CHUNK,), jnp.int32),
            pltpu.VMEM((CHUNK, D), jnp.float32),
            pltpu.VMEM((16, D), jnp.float32),
            pltpu.VMEM_SHARED((R, D), jnp.float32),
        ],
    )(_sc_agg_body)
    return f(x, src, dst)


BLK = 1024


def _tc_mlp_body(x_ref, agg_ref, nid_ref,
                 w1_ref, b1_ref, w2_ref, b2_ref,
                 wi1_ref, bi1_ref, wi2_ref, bi2_ref,
                 out_ref):
    i = pl.program_id(0)
    h = x_ref[...] + agg_ref[0] + agg_ref[1]
    h1 = jnp.maximum(
        jnp.dot(h, w1_ref[...], preferred_element_type=jnp.float32)
        + b1_ref[...], 0.0)
    o1 = jnp.dot(h1, w2_ref[...], preferred_element_type=jnp.float32) + b2_ref[...]
    h2 = jnp.maximum(
        jnp.dot(h, wi1_ref[...], preferred_element_type=jnp.float32)
        + bi1_ref[...], 0.0)
    o2 = jnp.dot(h2, wi2_ref[...], preferred_element_type=jnp.float32) + bi2_ref[...]

    nid = nid_ref[...]
    lo = jnp.bitwise_and(nid, 127)
    hi = lax.shift_right_logical(nid, 7)
    lane128 = lax.broadcasted_iota(jnp.int32, (128, 128), 1)
    lane8 = lax.broadcasted_iota(jnp.int32, (128, 8), 1)
    cnt = jnp.zeros((128, 8), jnp.float32)
    for t in range(NID_TOT // 128):
        loc = lo[:, t:t + 1]
        hic = hi[:, t:t + 1]
        L = (loc == lane128).astype(jnp.float32)
        H = (hic == i * 8 + lane8).astype(jnp.float32)
        cnt = cnt + lax.dot_general(
            L, H, (((0,), (0,)), ((), ())),
            preferred_element_type=jnp.float32)
    c = jnp.concatenate([cnt[:, k:k + 1] for k in range(BLK // 128)], axis=0)
    out_ref[...] = o1 + c * o2


def _tc_mlp(x_pad, agg_p, nid_t, W1t, b1, W2t, b2, Wi1t, bi1, Wi2t, bi2):
    grid = (R // BLK,)
    wspec = pl.BlockSpec((D, D), lambda i: (0, 0))
    bspec = pl.BlockSpec((1, D), lambda i: (0, 0))
    return pl.pallas_call(
        _tc_mlp_body,
        grid=grid,
        in_specs=[
            pl.BlockSpec((BLK, D), lambda i: (i, 0)),
            pl.BlockSpec((2, BLK, D), lambda i: (0, i, 0)),
            pl.BlockSpec((128, NID_TOT // 128), lambda i: (0, 0)),
            wspec, bspec, wspec, bspec,
            wspec, bspec, wspec, bspec,
        ],
        out_specs=pl.BlockSpec((BLK, D), lambda i: (i, 0)),
        out_shape=jax.ShapeDtypeStruct((R, D), jnp.float32),
    )(x_pad, agg_p, nid_t, W1t, b1, W2t, b2, Wi1t, bi1, Wi2t, bi2)


def kernel(x, edge_index, node_id, W1, b1, W2, b2, Wi1, bi1, Wi2, bi2):
    src = edge_index[0]
    dst = edge_index[1]
    agg_p = _sc_aggregate(x, src, dst)
    nid_t = jnp.concatenate(
        [node_id, jnp.full((NID_TOT - NID,), NID_PAD, jnp.int32)]
    ).reshape(NID_TOT // 128, 128).T
    x_pad = jnp.pad(x, ((0, R - N), (0, 0)))
    out = _tc_mlp(x_pad, agg_p, nid_t,
                  W1.T, b1.reshape(1, D), W2.T, b2.reshape(1, D),
                  Wi1.T, bi1.reshape(1, D), Wi2.T, bi2.reshape(1, D))
    return out[:N]
